# baseline (speedup 1.0000x reference)
import jax
import jax.numpy as jnp
from jax import lax
from jax.experimental import pallas as pl
from jax.experimental.pallas import tpu as pltpu
from jax.experimental.pallas import tpu_sc as plsc

_SMOOTHING = 0.1
_PAD = 0
_V = 32000
_ON = 1.0 - _SMOOTHING
_BASE = _SMOOTHING / (_V - 2)

_BB = 512
_VB = 16000

_NC = 2
_NS = 16
_NW = _NC * _NS


def _fill_body(tgt_ref, out_ref):
    j = pl.program_id(1)
    tgt = tgt_ref[...]
    rowv = jnp.where(tgt == _PAD, 0.0, _BASE)
    out_ref[...] = jnp.broadcast_to(rowv, (_BB, _VB))

    @pl.when(j == 0)
    def _zero_pad_col():
        out_ref[:, 0:1] = jnp.zeros((_BB, 1), jnp.float32)


def _tc_fill(tgt_ids):
    B = tgt_ids.shape[0]
    grid = (B // _BB, _V // _VB)
    return pl.pallas_call(
        _fill_body,
        grid=grid,
        in_specs=[pl.BlockSpec((_BB, 1), lambda i, j: (i, 0))],
        out_specs=pl.BlockSpec((_BB, _VB), lambda i, j: (i, j)),
        out_shape=jax.ShapeDtypeStruct((B, _V), jnp.float32),
        compiler_params=pltpu.CompilerParams(
            dimension_semantics=("parallel", "parallel"),
            vmem_limit_bytes=100 * 1024 * 1024,
        ),
    )(tgt_ids)


def _sc_scatter_body(tgt_hbm, out_ref, tgt_v, chunks_v, sem):
    per_w = tgt_hbm.shape[0] // _NW
    wid = lax.axis_index("s") * _NC + lax.axis_index("c")
    base = wid * per_w
    pltpu.sync_copy(tgt_hbm.at[pl.ds(base, per_w)], tgt_v)
    lanes = lax.iota(jnp.int32, 16)

    def build_and_fire(i, carry):
        t16 = tgt_v[pl.ds((i // 16) * 16, 16)]
        tgt_b = jnp.sum(jnp.where(lanes == i % 16, t16, 0))
        tc = tgt_b // 128
        tl = tgt_b % 128
        is_pad = tgt_b == _PAD
        for c in range(8):
            ln = lanes + (c * 16)
            v = jnp.where(ln == tl, jnp.float32(_ON), jnp.float32(_BASE))
            dead = is_pad | ((ln == 0) & (tc == 0))
            chunks_v[pl.ds(i * 128 + c * 16, 16)] = jnp.where(
                dead, jnp.float32(0.0), v)
        pltpu.async_copy(
            chunks_v.at[pl.ds(i * 128, 128)],
            out_ref.at[base + i, pl.ds(tc * 128, 128)],
            sem)
        return carry

    lax.fori_loop(0, per_w, build_and_fire, 0)

    drain = pltpu.make_async_copy(
        out_ref.at[0, pl.ds(0, 128)], tgt_v_dummy_dst(chunks_v), sem)
    lax.fori_loop(0, per_w, lambda i, c: drain_wait(drain, c), 0)


def tgt_v_dummy_dst(chunks_v):
    return chunks_v.at[pl.ds(0, 128)]


def drain_wait(drain, carry):
    drain.wait()
    return carry


def _sc_scatter(tgt_flat, out_ref2d):
    B = tgt_flat.shape[0]
    per_w = B // _NW
    mesh = plsc.VectorSubcoreMesh(
        core_axis_name="c", subcore_axis_name="s",
        num_cores=_NC, num_subcores=_NS)
    fn = pl.kernel(
        _sc_scatter_body,
        out_type=(),
        mesh=mesh,
        scratch_types=[
            pltpu.VMEM((per_w,), jnp.int32),
            pltpu.VMEM((per_w * 128,), jnp.float32),
            pltpu.SemaphoreType.DMA,
        ],
        compiler_params=pltpu.CompilerParams(needs_layout_passes=False),
    )
    fn(tgt_flat, out_ref2d)


def kernel(tgt_ids):
    B = tgt_ids.shape[0]
    filled = _tc_fill(tgt_ids)
    out_ref = jax.new_ref(filled)
    _sc_scatter(tgt_ids.reshape(B), out_ref)
    return jax.freeze(out_ref)


# speedup vs baseline: 1.0064x; 1.0064x over previous
import jax
import jax.numpy as jnp
from jax import lax
from jax.experimental import pallas as pl
from jax.experimental.pallas import tpu as pltpu
from jax.experimental.pallas import tpu_sc as plsc

_SMOOTHING = 0.1
_PAD = 0
_V = 32000
_ON = 1.0 - _SMOOTHING
_BASE = _SMOOTHING / (_V - 2)

_BB = 2048
_VB = 3200

_NC = 2
_NS = 16
_NW = _NC * _NS


def _fill_body(tgt_ref, out_ref):
    j = pl.program_id(1)
    tgt = tgt_ref[...]
    rowv = jnp.where(tgt == _PAD, 0.0, _BASE)
    out_ref[...] = jnp.broadcast_to(rowv, (_BB, _VB))

    @pl.when(j == 0)
    def _zero_pad_col():
        out_ref[:, 0:1] = jnp.zeros((_BB, 1), jnp.float32)


def _tc_fill(tgt_ids):
    B = tgt_ids.shape[0]
    grid = (B // _BB, _V // _VB)
    return pl.pallas_call(
        _fill_body,
        grid=grid,
        in_specs=[pl.BlockSpec((_BB, 1), lambda i, j: (i, 0))],
        out_specs=pl.BlockSpec((_BB, _VB), lambda i, j: (i, j)),
        out_shape=jax.ShapeDtypeStruct((B, _V), jnp.float32),
        compiler_params=pltpu.CompilerParams(
            dimension_semantics=("parallel", "parallel"),
            vmem_limit_bytes=100 * 1024 * 1024,
        ),
    )(tgt_ids)


def _sc_scatter_body(tgt_hbm, out_ref, tgt_v, chunks_v, sem):
    per_w = tgt_hbm.shape[0] // _NW
    wid = lax.axis_index("s") * _NC + lax.axis_index("c")
    base = wid * per_w
    pltpu.sync_copy(tgt_hbm.at[pl.ds(base, per_w)], tgt_v)
    lanes = lax.iota(jnp.int32, 16)

    def build_and_fire(i, carry):
        t16 = tgt_v[pl.ds((i // 16) * 16, 16)]
        tgt_b = jnp.sum(jnp.where(lanes == i % 16, t16, 0))
        tc = tgt_b // 128
        tl = tgt_b % 128
        is_pad = tgt_b == _PAD
        for c in range(8):
            ln = lanes + (c * 16)
            v = jnp.where(ln == tl, jnp.float32(_ON), jnp.float32(_BASE))
            dead = is_pad | ((ln == 0) & (tc == 0))
            chunks_v[pl.ds(i * 128 + c * 16, 16)] = jnp.where(
                dead, jnp.float32(0.0), v)
        pltpu.async_copy(
            chunks_v.at[pl.ds(i * 128, 128)],
            out_ref.at[base + i, pl.ds(tc * 128, 128)],
            sem)
        return carry

    lax.fori_loop(0, per_w, build_and_fire, 0)

    drain = pltpu.make_async_copy(
        out_ref.at[0, pl.ds(0, 128)], tgt_v_dummy_dst(chunks_v), sem)
    lax.fori_loop(0, per_w, lambda i, c: drain_wait(drain, c), 0)


def tgt_v_dummy_dst(chunks_v):
    return chunks_v.at[pl.ds(0, 128)]


def drain_wait(drain, carry):
    drain.wait()
    return carry


def _sc_scatter(tgt_flat, out_ref2d):
    B = tgt_flat.shape[0]
    per_w = B // _NW
    mesh = plsc.VectorSubcoreMesh(
        core_axis_name="c", subcore_axis_name="s",
        num_cores=_NC, num_subcores=_NS)
    fn = pl.kernel(
        _sc_scatter_body,
        out_type=(),
        mesh=mesh,
        scratch_types=[
            pltpu.VMEM((per_w,), jnp.int32),
            pltpu.VMEM((per_w * 128,), jnp.float32),
            pltpu.SemaphoreType.DMA,
        ],
        compiler_params=pltpu.CompilerParams(needs_layout_passes=False),
    )
    fn(tgt_flat, out_ref2d)


def kernel(tgt_ids):
    B = tgt_ids.shape[0]
    filled = _tc_fill(tgt_ids)
    out_ref = jax.new_ref(filled)
    _sc_scatter(tgt_ids.reshape(B), out_ref)
    return jax.freeze(out_ref)


# speedup vs baseline: 1.0147x; 1.0082x over previous
import jax
import jax.numpy as jnp
from jax import lax
from jax.experimental import pallas as pl
from jax.experimental.pallas import tpu as pltpu
from jax.experimental.pallas import tpu_sc as plsc

_SMOOTHING = 0.1
_PAD = 0
_V = 32000
_ON = 1.0 - _SMOOTHING
_BASE = _SMOOTHING / (_V - 2)

_BB = 1024
_VB = 6400

_NC = 2
_NS = 16
_NW = _NC * _NS


def _fill_body(tgt_ref, out_ref):
    j = pl.program_id(1)
    tgt = tgt_ref[...]
    rowv = jnp.where(tgt == _PAD, 0.0, _BASE)
    out_ref[...] = jnp.broadcast_to(rowv, (_BB, _VB))

    @pl.when(j == 0)
    def _zero_pad_col():
        out_ref[:, 0:1] = jnp.zeros((_BB, 1), jnp.float32)


def _tc_fill(tgt_ids):
    B = tgt_ids.shape[0]
    grid = (B // _BB, _V // _VB)
    return pl.pallas_call(
        _fill_body,
        grid=grid,
        in_specs=[pl.BlockSpec((_BB, 1), lambda i, j: (i, 0))],
        out_specs=pl.BlockSpec((_BB, _VB), lambda i, j: (i, j)),
        out_shape=jax.ShapeDtypeStruct((B, _V), jnp.float32),
        compiler_params=pltpu.CompilerParams(
            dimension_semantics=("parallel", "parallel"),
        ),
    )(tgt_ids)


def _sc_scatter_body(tgt_hbm, out_ref, tgt_v, chunks_v, sem):
    per_w = tgt_hbm.shape[0] // _NW
    wid = lax.axis_index("s") * _NC + lax.axis_index("c")
    base = wid * per_w
    pltpu.sync_copy(tgt_hbm.at[pl.ds(base, per_w)], tgt_v)
    lanes = lax.iota(jnp.int32, 16)

    def build_and_fire(i, carry):
        t16 = tgt_v[pl.ds((i // 16) * 16, 16)]
        tgt_b = jnp.sum(jnp.where(lanes == i % 16, t16, 0))
        tc = tgt_b // 128
        tl = tgt_b % 128
        is_pad = tgt_b == _PAD
        for c in range(8):
            ln = lanes + (c * 16)
            v = jnp.where(ln == tl, jnp.float32(_ON), jnp.float32(_BASE))
            dead = is_pad | ((ln == 0) & (tc == 0))
            chunks_v[pl.ds(i * 128 + c * 16, 16)] = jnp.where(
                dead, jnp.float32(0.0), v)
        pltpu.async_copy(
            chunks_v.at[pl.ds(i * 128, 128)],
            out_ref.at[base + i, pl.ds(tc * 128, 128)],
            sem)
        return carry

    lax.fori_loop(0, per_w, build_and_fire, 0)

    drain = pltpu.make_async_copy(
        out_ref.at[0, pl.ds(0, 128)], chunks_v.at[pl.ds(0, 128)], sem)

    def _wait_one(i, carry):
        drain.wait()
        return carry

    lax.fori_loop(0, per_w, _wait_one, 0)


def _sc_scatter(tgt_flat, out_ref2d):
    B = tgt_flat.shape[0]
    per_w = B // _NW
    mesh = plsc.VectorSubcoreMesh(
        core_axis_name="c", subcore_axis_name="s",
        num_cores=_NC, num_subcores=_NS)
    fn = pl.kernel(
        _sc_scatter_body,
        out_type=(),
        mesh=mesh,
        scratch_types=[
            pltpu.VMEM((per_w,), jnp.int32),
            pltpu.VMEM((per_w * 128,), jnp.float32),
            pltpu.SemaphoreType.DMA,
        ],
        compiler_params=pltpu.CompilerParams(needs_layout_passes=False),
    )
    fn(tgt_flat, out_ref2d)


def kernel(tgt_ids):
    B = tgt_ids.shape[0]
    filled = _tc_fill(tgt_ids)
    out_ref = jax.new_ref(filled)
    _sc_scatter(tgt_ids.reshape(B), out_ref)
    return jax.freeze(out_ref)
